# Initial kernel scaffold; baseline (speedup 1.0000x reference)
import jax
import jax.numpy as jnp
from jax.experimental import pallas as pl


def kernel(x, center):
    raise NotImplementedError("write your pallas kernel here")


# speedup vs baseline: 120.6596x; 120.6596x over previous
import functools

import jax
import jax.numpy as jnp
from jax import lax
from jax.experimental import pallas as pl
from jax.experimental.pallas import tpu as pltpu
from jax.experimental.pallas import tpu_sc as plsc

_ROWS, _COLS = 100000, 256
_SEG = 64
_RPR = _COLS // _SEG
_NC, _NS, _L = 2, 16, 16
_NW = _NC * _NS
_CHUNK = 128
_NCHUNK = 5
_PER_W = _CHUNK * _NCHUNK
_NPAD = _NW * _PER_W

_mesh = plsc.VectorSubcoreMesh(core_axis_name="c", subcore_axis_name="s")


@functools.partial(
    pl.kernel,
    out_type=(),
    mesh=_mesh,
    compiler_params=pltpu.CompilerParams(use_tc_tiling_on_sc=False),
    scratch_types=[
        pltpu.VMEM((_NCHUNK, _CHUNK), jnp.int32),
        pltpu.VMEM((_PER_W, _SEG), jnp.float32),
        pltpu.SemaphoreType.DMA,
        pltpu.SemaphoreType.DMA,
    ],
)
def _sc_update(x4, idx, out, idx_v, rows_v, gsem, ssem):
    wid = lax.axis_index("s") * _NC + lax.axis_index("c")
    pltpu.sync_copy(idx.at[wid], idx_v)

    for j in range(_NCHUNK):
        for k in range(_CHUNK // _L):
            sl = pl.ds(k * _L, _L)
            idx_v[j, sl] = idx_v[j, sl] * _RPR

    for j in range(_NCHUNK):
        pltpu.make_async_copy(
            x4.at[idx_v.at[j]], rows_v.at[pl.ds(j * _CHUNK, _CHUNK)], gsem
        ).start()
    for j in range(_NCHUNK):
        pltpu.make_async_copy(
            x4.at[idx_v.at[j]], rows_v.at[pl.ds(j * _CHUNK, _CHUNK)], gsem
        ).wait()

    def body(i, carry):
        for k in range(_SEG // _L):
            sl = pl.ds(k * _L, _L)
            rows_v[i, sl] = jnp.minimum(rows_v[i, sl] + 0.5, 1.0)
        return carry

    lax.fori_loop(0, _PER_W, body, 0)

    for j in range(_NCHUNK):
        pltpu.make_async_copy(
            rows_v.at[pl.ds(j * _CHUNK, _CHUNK)], out.at[idx_v.at[j]], ssem
        ).start()
    for j in range(_NCHUNK):
        pltpu.make_async_copy(
            rows_v.at[pl.ds(j * _CHUNK, _CHUNK)], out.at[idx_v.at[j]], ssem
        ).wait()


def kernel(x, center):
    x4 = x.reshape(_ROWS * _RPR, _SEG)
    idx = jnp.pad(center, (0, _NPAD - center.shape[0]), mode="edge")
    idx = idx.reshape(_NW, _NCHUNK, _CHUNK)
    out_ref = jax.new_ref(x4)
    _sc_update(x4, idx, out_ref)
    return out_ref[...].reshape(_ROWS, _COLS)


# speedup vs baseline: 2117.8501x; 17.5523x over previous
import functools

import jax
import jax.numpy as jnp
from jax import lax
from jax.experimental import pallas as pl
from jax.experimental.pallas import tpu as pltpu
from jax.experimental.pallas import tpu_sc as plsc

_ROWS, _COLS = 100000, 256
_SEG = 64
_RPR = _COLS // _SEG
_NC, _NS, _L = 2, 16, 16
_NW = _NC * _NS
_CHUNK = 128
_NCHT = 160
_NPAD = _NCHT * _CHUNK
_NCH0, _NCH1 = 8, 2
_ROWS_V = _NCH0 * _CHUNK

_mesh = plsc.VectorSubcoreMesh(core_axis_name="c", subcore_axis_name="s")

_SLAB = (_ROWS * _RPR) // _NW
_CROWS = 625
_NCOPY = _SLAB // _CROWS
_NBUF = 3


@functools.partial(
    pl.kernel,
    out_type=jax.ShapeDtypeStruct((_ROWS * _RPR, _SEG), jnp.float32),
    mesh=_mesh,
    compiler_params=pltpu.CompilerParams(use_tc_tiling_on_sc=False),
    scratch_types=[
        pltpu.VMEM((_NBUF, _CROWS, _SEG), jnp.float32),
        pltpu.SemaphoreType.DMA,
        pltpu.SemaphoreType.DMA,
    ],
)
def _sc_copy(x4, out, buf, rsem, wsem):
    wid = lax.axis_index("s") * _NC + lax.axis_index("c")
    base = wid * _SLAB

    def rd(c):
        return pltpu.make_async_copy(
            x4.at[pl.ds(base + c * _CROWS, _CROWS)], buf.at[c % _NBUF], rsem
        )

    def wr(c):
        return pltpu.make_async_copy(
            buf.at[c % _NBUF], out.at[pl.ds(base + c * _CROWS, _CROWS)], wsem
        )

    for c in range(_NBUF):
        rd(c).start()
    for c in range(_NCOPY):
        rd(c).wait()
        wr(c).start()
        if c + _NBUF < _NCOPY:
            wr(c).wait()
            rd(c + _NBUF).start()
    for c in range(_NCOPY - _NBUF, _NCOPY):
        wr(c).wait()


@functools.partial(
    pl.kernel,
    out_type=(),
    mesh=_mesh,
    compiler_params=pltpu.CompilerParams(use_tc_tiling_on_sc=False),
    scratch_types=[
        pltpu.VMEM((_NCH0, _CHUNK), jnp.int32),
        pltpu.VMEM((_ROWS_V, _SEG), jnp.float32),
        pltpu.SemaphoreType.DMA,
        pltpu.SemaphoreType.DMA,
    ],
)
def _sc_update(x4, idx, out, idx_v, rows_v, gsem, ssem):
    c = lax.axis_index("c")
    s = lax.axis_index("s")

    def do_update(nch, q0):
        pltpu.sync_copy(idx.at[pl.ds(q0, nch)], idx_v.at[pl.ds(0, nch)])

        for j in range(nch):
            for k in range(_CHUNK // _L):
                sl = pl.ds(k * _L, _L)
                r = idx_v[j, sl]
                idx_v[j, sl] = ((r >> 3) << 5) + ((r & 7) << 1)

        for j in range(nch):
            pltpu.make_async_copy(
                x4.at[idx_v.at[j]], rows_v.at[pl.ds(j * _CHUNK, _CHUNK)], gsem
            ).start()
        for j in range(nch):
            pltpu.make_async_copy(
                x4.at[idx_v.at[j]], rows_v.at[pl.ds(j * _CHUNK, _CHUNK)], gsem
            ).wait()

        def body(i, carry):
            for k in range(_SEG // _L):
                sl = pl.ds(k * _L, _L)
                rows_v[i, sl] = jnp.minimum(rows_v[i, sl] + 0.5, 1.0)
            return carry

        lax.fori_loop(0, nch * _CHUNK, body, 0)

        for j in range(nch):
            pltpu.make_async_copy(
                rows_v.at[pl.ds(j * _CHUNK, _CHUNK)], out.at[idx_v.at[j]], ssem
            ).start()
        for j in range(nch):
            pltpu.make_async_copy(
                rows_v.at[pl.ds(j * _CHUNK, _CHUNK)], out.at[idx_v.at[j]], ssem
            ).wait()

    @pl.when(c == 0)
    def _():
        do_update(_NCH0, s * _NCH0)

    @pl.when(c == 1)
    def _():
        do_update(_NCH1, _NS * _NCH0 + s * _NCH1)


def kernel(x, center):
    xb = (
        x.reshape(_ROWS // 8, 8, _COLS // 128, 128)
        .transpose(0, 2, 1, 3)
        .reshape(_ROWS * _RPR, _SEG)
    )
    idx = jnp.pad(center, (0, _NPAD - center.shape[0]), mode="edge")
    idx = idx.reshape(_NCHT, _CHUNK)
    out = _sc_copy(xb)
    out_ref = jax.new_ref(out)
    _sc_update(xb, idx, out_ref)
    o = out_ref[...]
    return (
        o.reshape(_ROWS // 8, _COLS // 128, 8, 128)
        .transpose(0, 2, 1, 3)
        .reshape(_ROWS, _COLS)
    )
